# baseline (speedup 1.0000x reference)
import functools

import jax
import jax.numpy as jnp
from jax import lax
from jax.experimental import pallas as pl
from jax.experimental.pallas import tpu as pltpu
from jax.experimental.pallas import tpu_sc as plsc

B = 16384
D = 64
NC = 2
NS = 16
NW = NC * NS
BPW = B // NW
IDX_CHUNK = 128
NCHUNK = BPW // IDX_CHUNK

_F32 = jnp.float32


def _sc_body(idx_hbm, table_hbm, out_hbm, idx_v, rows_v, sem):
    wid = lax.axis_index("s") * NC + lax.axis_index("c")
    base = wid * BPW

    pltpu.sync_copy(idx_hbm.at[wid], idx_v)

    copies = [
        pltpu.async_copy(
            table_hbm.at[idx_v.at[j]],
            rows_v.at[pl.ds(j * IDX_CHUNK, IDX_CHUNK)],
            sem,
        )
        for j in range(NCHUNK)
    ]
    for c in copies:
        c.wait()

    half = jnp.full((16,), -0.5, dtype=_F32)
    three_half = jnp.full((16,), 1.5, dtype=_F32)
    one = jnp.full((16,), 1.0, dtype=_F32)
    magic = jnp.full((16,), 0x5F3759DF, dtype=jnp.int32)
    out_scale = jnp.full((16,), 0.99, dtype=_F32)
    lanes = jnp.arange(16, dtype=jnp.int32)
    perms = [lanes ^ sh for sh in (8, 4, 2, 1)]
    dnums = lax.GatherDimensionNumbers(
        offset_dims=(), collapsed_slice_dims=(0,), start_index_map=(0,))

    def shuffle(x, idx):
        return lax.gather(x, idx[:, None], dnums, (1,),
                          mode=lax.GatherScatterMode.PROMISE_IN_BOUNDS)

    def row(i, carry):
        x0 = rows_v[i, pl.ds(0, 16)]
        x1 = rows_v[i, pl.ds(16, 16)]
        x2 = rows_v[i, pl.ds(32, 16)]
        x3 = rows_v[i, pl.ds(48, 16)]
        acc = x0 * x0 + x1 * x1 + x2 * x2 + x3 * x3
        for p in perms:
            acc = acc + shuffle(acc, p)
        nsq = jnp.maximum(acc, one)
        y = plsc.bitcast(magic - (plsc.bitcast(nsq, jnp.int32) >> 1), _F32)
        h = nsq * half
        y = y * (three_half + h * y * y)
        y = y * (three_half + h * y * y)
        s = y * out_scale
        rows_v[i, pl.ds(0, 16)] = x0 * s
        rows_v[i, pl.ds(16, 16)] = x1 * s
        rows_v[i, pl.ds(32, 16)] = x2 * s
        rows_v[i, pl.ds(48, 16)] = x3 * s
        return carry

    lax.fori_loop(0, BPW, row, 0)

    pltpu.sync_copy(rows_v, out_hbm.at[pl.ds(base, BPW)])


def kernel(taxon_ids, table):
    idx = taxon_ids.astype(jnp.int32).reshape(NW, NCHUNK, IDX_CHUNK)
    k = pl.kernel(
        _sc_body,
        out_type=jax.ShapeDtypeStruct((B, D), _F32),
        mesh=plsc.VectorSubcoreMesh(core_axis_name="c", subcore_axis_name="s"),
        compiler_params=pltpu.CompilerParams(
            needs_layout_passes=False, use_tc_tiling_on_sc=False),
        scratch_types=[
            pltpu.VMEM((NCHUNK, IDX_CHUNK), jnp.int32),
            pltpu.VMEM((BPW, D), _F32),
            pltpu.SemaphoreType.DMA,
        ],
    )
    return k(idx, table)


# speedup vs baseline: 1.1557x; 1.1557x over previous
import functools

import jax
import jax.numpy as jnp
from jax import lax
from jax.experimental import pallas as pl
from jax.experimental.pallas import tpu as pltpu
from jax.experimental.pallas import tpu_sc as plsc

B = 16384
D = 64
NC = 2
NS = 16
NW = NC * NS
BPW = B // NW
IDX_CHUNK = 128
NCHUNK = BPW // IDX_CHUNK

_F32 = jnp.float32


def _sc_body(idx_hbm, table_hbm, out_hbm, idx_v, rows_v, gsem, osem):
    wid = lax.axis_index("s") * NC + lax.axis_index("c")
    base = wid * BPW

    pltpu.sync_copy(idx_hbm.at[wid], idx_v)

    gathers = [
        pltpu.async_copy(
            table_hbm.at[idx_v.at[j]],
            rows_v.at[pl.ds(j * IDX_CHUNK, IDX_CHUNK)],
            gsem,
        )
        for j in range(NCHUNK)
    ]

    scale99 = jnp.full((16,), 0.99, dtype=_F32)
    m = jnp.zeros((16,), dtype=_F32)
    out_copies = []

    for j in range(NCHUNK):
        gathers[j].wait()

        @plsc.parallel_loop(j * IDX_CHUNK, (j + 1) * IDX_CHUNK, carry=m,
                            unroll=4)
        def _scale_row(i, mc):
            a0 = rows_v[i, pl.ds(0, 16)]
            a1 = rows_v[i, pl.ds(16, 16)]
            a2 = rows_v[i, pl.ds(32, 16)]
            a3 = rows_v[i, pl.ds(48, 16)]
            mc = jnp.maximum(jnp.maximum(mc, a0 * a0),
                             jnp.maximum(a1 * a1, a2 * a2))
            mc = jnp.maximum(mc, a3 * a3)
            rows_v[i, pl.ds(0, 16)] = a0 * scale99
            rows_v[i, pl.ds(16, 16)] = a1 * scale99
            rows_v[i, pl.ds(32, 16)] = a2 * scale99
            rows_v[i, pl.ds(48, 16)] = a3 * scale99
            return mc

        m = _scale_row
        out_copies.append(pltpu.async_copy(
            rows_v.at[pl.ds(j * IDX_CHUNK, IDX_CHUNK)],
            out_hbm.at[pl.ds(base + j * IDX_CHUNK, IDX_CHUNK)],
            osem,
        ))

    for c in out_copies:
        c.wait()

    g = jnp.max(m)

    def _fixup(_):
        half = jnp.full((16,), -0.5, dtype=_F32)
        three_half = jnp.full((16,), 1.5, dtype=_F32)
        one = jnp.full((16,), 1.0, dtype=_F32)
        inv9801 = jnp.full((16,), 1.0 / (0.99 * 0.99), dtype=_F32)
        magic = jnp.full((16,), 0x5F3759DF, dtype=jnp.int32)

        def row(i, carry):
            y0 = rows_v[i, pl.ds(0, 16)]
            y1 = rows_v[i, pl.ds(16, 16)]
            y2 = rows_v[i, pl.ds(32, 16)]
            y3 = rows_v[i, pl.ds(48, 16)]
            acc = y0 * y0 + y1 * y1 + y2 * y2 + y3 * y3
            nsq = jnp.full((16,), jnp.sum(acc), dtype=_F32) * inv9801
            nsq = jnp.maximum(nsq, one)
            w = plsc.bitcast(magic - (plsc.bitcast(nsq, jnp.int32) >> 1),
                             _F32)
            h = nsq * half
            w = w * (three_half + h * w * w)
            w = w * (three_half + h * w * w)
            rows_v[i, pl.ds(0, 16)] = y0 * w
            rows_v[i, pl.ds(16, 16)] = y1 * w
            rows_v[i, pl.ds(32, 16)] = y2 * w
            rows_v[i, pl.ds(48, 16)] = y3 * w
            return carry

        lax.fori_loop(0, BPW, row, 0)
        pltpu.sync_copy(rows_v, out_hbm.at[pl.ds(base, BPW)])

    lax.cond(g * 64.0 > 1.0, _fixup, lambda _: None, 0)


def kernel(taxon_ids, table):
    idx = taxon_ids.astype(jnp.int32).reshape(NW, NCHUNK, IDX_CHUNK)
    k = pl.kernel(
        _sc_body,
        out_type=jax.ShapeDtypeStruct((B, D), _F32),
        mesh=plsc.VectorSubcoreMesh(core_axis_name="c", subcore_axis_name="s"),
        compiler_params=pltpu.CompilerParams(
            needs_layout_passes=False, use_tc_tiling_on_sc=False),
        scratch_types=[
            pltpu.VMEM((NCHUNK, IDX_CHUNK), jnp.int32),
            pltpu.VMEM((BPW, D), _F32),
            pltpu.SemaphoreType.DMA,
            pltpu.SemaphoreType.DMA,
        ],
    )
    return k(idx, table)


# speedup vs baseline: 1.3152x; 1.1380x over previous
import jax
import jax.numpy as jnp
from jax import lax
from jax.experimental import pallas as pl
from jax.experimental.pallas import tpu as pltpu
from jax.experimental.pallas import tpu_sc as plsc

B = 16384
D = 64
NC = 2
NS = 16
NW = NC * NS
BPW = B // NW
IDX_CHUNK = 128
NCHUNK = BPW // IDX_CHUNK
PITCH = 517

_F32 = jnp.float32


def _sc_body(idx_hbm, table_hbm, out_hbm, idx_v, rows_v, trans_v, gsem, osem):
    wid = lax.axis_index("s") * NC + lax.axis_index("c")

    pltpu.sync_copy(idx_hbm.at[wid], idx_v)

    gathers = [
        pltpu.async_copy(
            table_hbm.at[idx_v.at[j]],
            rows_v.at[pl.ds(j * IDX_CHUNK, IDX_CHUNK)],
            gsem,
        )
        for j in range(NCHUNK)
    ]

    lanes = jnp.arange(16, dtype=jnp.int32)
    sub_idx = [(16 * c + lanes) // 8 for c in range(NCHUNK)]
    din_idx = [(16 * c + lanes) % 8 for c in range(NCHUNK)]
    scale99 = jnp.full((16,), 0.99, dtype=_F32)
    m = jnp.zeros((16,), dtype=_F32)
    out_copies = []

    def scatter_row(i, vals):
        col = jnp.full((16,), i, dtype=jnp.int32)
        for c in range(NCHUNK):
            plsc.store_scatter(trans_v, [sub_idx[c], din_idx[c], col], vals[c])

    for j in range(NCHUNK):
        gathers[j].wait()

        @plsc.parallel_loop(j * IDX_CHUNK, (j + 1) * IDX_CHUNK, carry=m,
                            unroll=4)
        def _scale_row(i, mc):
            a0 = rows_v[i, pl.ds(0, 16)]
            a1 = rows_v[i, pl.ds(16, 16)]
            a2 = rows_v[i, pl.ds(32, 16)]
            a3 = rows_v[i, pl.ds(48, 16)]
            mc = jnp.maximum(jnp.maximum(mc, a0 * a0),
                             jnp.maximum(a1 * a1, a2 * a2))
            mc = jnp.maximum(mc, a3 * a3)
            scatter_row(i, [a0 * scale99, a1 * scale99,
                            a2 * scale99, a3 * scale99])
            return mc

        m = _scale_row
        out_copies.append(pltpu.async_copy(
            trans_v.at[:, :, pl.ds(j * IDX_CHUNK, IDX_CHUNK)],
            out_hbm.at[:, wid * NCHUNK + j],
            osem,
        ))

    for c in out_copies:
        c.wait()

    g = jnp.max(m)

    def _fixup(_):
        half = jnp.full((16,), -0.5, dtype=_F32)
        three_half = jnp.full((16,), 1.5, dtype=_F32)
        one = jnp.full((16,), 1.0, dtype=_F32)
        magic = jnp.full((16,), 0x5F3759DF, dtype=jnp.int32)

        def row(i, carry):
            x0 = rows_v[i, pl.ds(0, 16)]
            x1 = rows_v[i, pl.ds(16, 16)]
            x2 = rows_v[i, pl.ds(32, 16)]
            x3 = rows_v[i, pl.ds(48, 16)]
            acc = x0 * x0 + x1 * x1 + x2 * x2 + x3 * x3
            nsq = jnp.maximum(jnp.full((16,), jnp.sum(acc), dtype=_F32), one)
            w = plsc.bitcast(magic - (plsc.bitcast(nsq, jnp.int32) >> 1),
                             _F32)
            h = nsq * half
            w = w * (three_half + h * w * w)
            w = w * (three_half + h * w * w)
            w = w * scale99
            scatter_row(i, [x0 * w, x1 * w, x2 * w, x3 * w])
            return carry

        lax.fori_loop(0, BPW, row, 0)
        for j in range(NCHUNK):
            pltpu.sync_copy(
                trans_v.at[:, :, pl.ds(j * IDX_CHUNK, IDX_CHUNK)],
                out_hbm.at[:, wid * NCHUNK + j],
            )

    lax.cond(g * 64.0 > 1.0, _fixup, lambda _: None, 0)


def kernel(taxon_ids, table):
    idx = taxon_ids.astype(jnp.int32).reshape(NW, NCHUNK, IDX_CHUNK)
    k = pl.kernel(
        _sc_body,
        out_type=jax.ShapeDtypeStruct((8, B // 128, 8, 128), _F32),
        mesh=plsc.VectorSubcoreMesh(core_axis_name="c", subcore_axis_name="s"),
        compiler_params=pltpu.CompilerParams(
            needs_layout_passes=False, use_tc_tiling_on_sc=False),
        scratch_types=[
            pltpu.VMEM((NCHUNK, IDX_CHUNK), jnp.int32),
            pltpu.VMEM((BPW, D), _F32),
            pltpu.VMEM((8, 8, PITCH), _F32),
            pltpu.SemaphoreType.DMA,
            pltpu.SemaphoreType.DMA,
        ],
    )
    out4 = k(idx, table)
    return out4.transpose(0, 2, 1, 3).reshape(D, B).T


# speedup vs baseline: 2.5864x; 1.9665x over previous
import jax
import jax.numpy as jnp
from jax import lax
from jax.experimental import pallas as pl
from jax.experimental.pallas import tpu as pltpu
from jax.experimental.pallas import tpu_sc as plsc

B = 16384
D = 64
V = 100000
NC = 2
NS = 16
NW = NC * NS
NPASS = D // NW
OCHUNK = 4096
NOCHUNK = B // OCHUNK

_F32 = jnp.float32


def _sc_body(tab_hbm, idx_hbm, out_hbm, flags_hbm, idx_v, trow_v, ob_v,
             fl_v):
    wid = lax.axis_index("s") * NC + lax.axis_index("c")

    pltpu.sync_copy(idx_hbm, idx_v)

    scale99 = jnp.full((16,), 0.99, dtype=_F32)
    zero = jnp.zeros((16,), dtype=_F32)
    m = zero

    for p in range(NPASS):
        d = p * NW + wid
        pltpu.sync_copy(tab_hbm.at[d], trow_v)

        for cb in range(NOCHUNK):

            @plsc.parallel_loop(0, OCHUNK // 16, carry=m, unroll=4)
            def _serve(k, mc):
                iv = idx_v[pl.ds(cb * OCHUNK + 16 * k, 16)]
                g = plsc.load_gather(trow_v, [iv])
                mc = jnp.maximum(mc, g * g)
                ob_v[pl.ds(16 * k, 16)] = g * scale99
                return mc

            m = _serve
            pltpu.sync_copy(ob_v, out_hbm.at[d, pl.ds(cb * OCHUNK, OCHUNK)])

    for c in range(8):
        fl_v[pl.ds(16 * c, 16)] = m if c == 0 else zero
    pltpu.sync_copy(fl_v, flags_hbm.at[wid])


def _fb_body(idx_hbm, table_hbm, out_hbm, idx_v, rows_v, sem):
    wid = lax.axis_index("s") * NC + lax.axis_index("c")
    bpw = B // NW
    pltpu.sync_copy(idx_hbm.at[wid], idx_v)
    copies = [
        pltpu.async_copy(
            table_hbm.at[idx_v.at[j]],
            rows_v.at[pl.ds(j * 128, 128)],
            sem,
        )
        for j in range(bpw // 128)
    ]
    for c in copies:
        c.wait()

    half = jnp.full((16,), -0.5, dtype=_F32)
    three_half = jnp.full((16,), 1.5, dtype=_F32)
    one = jnp.full((16,), 1.0, dtype=_F32)
    magic = jnp.full((16,), 0x5F3759DF, dtype=jnp.int32)
    scale99 = jnp.full((16,), 0.99, dtype=_F32)

    def row(i, carry):
        x0 = rows_v[i, pl.ds(0, 16)]
        x1 = rows_v[i, pl.ds(16, 16)]
        x2 = rows_v[i, pl.ds(32, 16)]
        x3 = rows_v[i, pl.ds(48, 16)]
        acc = x0 * x0 + x1 * x1 + x2 * x2 + x3 * x3
        nsq = jnp.maximum(jnp.full((16,), jnp.sum(acc), dtype=_F32), one)
        w = plsc.bitcast(magic - (plsc.bitcast(nsq, jnp.int32) >> 1), _F32)
        h = nsq * half
        w = w * (three_half + h * w * w)
        w = w * (three_half + h * w * w)
        w = w * scale99
        rows_v[i, pl.ds(0, 16)] = x0 * w
        rows_v[i, pl.ds(16, 16)] = x1 * w
        rows_v[i, pl.ds(32, 16)] = x2 * w
        rows_v[i, pl.ds(48, 16)] = x3 * w
        return carry

    lax.fori_loop(0, bpw, row, 0)
    pltpu.sync_copy(rows_v, out_hbm.at[pl.ds(wid * bpw, bpw)])


def _mesh():
    return plsc.VectorSubcoreMesh(core_axis_name="c", subcore_axis_name="s")


def _fallback(taxon_ids, table):
    idx = taxon_ids.astype(jnp.int32).reshape(NW, B // NW // 128, 128)
    k = pl.kernel(
        _fb_body,
        out_type=jax.ShapeDtypeStruct((B, D), _F32),
        mesh=_mesh(),
        compiler_params=pltpu.CompilerParams(
            needs_layout_passes=False, use_tc_tiling_on_sc=False),
        scratch_types=[
            pltpu.VMEM((B // NW // 128, 128), jnp.int32),
            pltpu.VMEM((B // NW, D), _F32),
            pltpu.SemaphoreType.DMA,
        ],
    )
    return k(idx, table)


def kernel(taxon_ids, table):
    table_t = table.T
    idx = taxon_ids.astype(jnp.int32)
    k = pl.kernel(
        _sc_body,
        out_type=(
            jax.ShapeDtypeStruct((D, B), _F32),
            jax.ShapeDtypeStruct((NW, 128), _F32),
        ),
        mesh=_mesh(),
        compiler_params=pltpu.CompilerParams(
            needs_layout_passes=False, use_tc_tiling_on_sc=True),
        scratch_types=[
            pltpu.VMEM((B,), jnp.int32),
            pltpu.VMEM((V,), _F32),
            pltpu.VMEM((OCHUNK,), _F32),
            pltpu.VMEM((128,), _F32),
        ],
    )
    out_t, flags = k(table_t, idx)
    tripped = jnp.max(flags) * 64.0 > 1.0
    return lax.cond(tripped,
                    lambda: _fallback(taxon_ids, table),
                    lambda: out_t.T)


# speedup vs baseline: 2.6395x; 1.0206x over previous
import jax
import jax.numpy as jnp
from jax import lax
from jax.experimental import pallas as pl
from jax.experimental.pallas import tpu as pltpu
from jax.experimental.pallas import tpu_sc as plsc

B = 16384
D = 64
V = 100000
NC = 2
NS = 16
NW = NC * NS
NPASS = D // NW
OCHUNK = 4096
NOCHUNK = B // OCHUNK

_F32 = jnp.float32


def _sc_body(tab_hbm, idx_hbm, out_hbm, flags_hbm, idx_v, trow_v, ob_v,
             isem, tsem, osem):
    wid = lax.axis_index("s") * NC + lax.axis_index("c")

    idx_cp = pltpu.async_copy(idx_hbm, idx_v, isem)
    def stage_row(d):
        return [pltpu.async_copy(tab_hbm.at[d], trow_v, tsem)]

    stage = stage_row(wid)
    idx_cp.wait()

    scale99 = jnp.full((16,), 0.99, dtype=_F32)
    zero = jnp.zeros((16,), dtype=_F32)
    m = zero
    out_cp = [None, None]

    for p in range(NPASS):
        d = p * NW + wid
        for s in stage:
            s.wait()

        for cb in range(NOCHUNK):
            buf = cb % 2
            if out_cp[buf] is not None:
                out_cp[buf].wait()

            @plsc.parallel_loop(0, OCHUNK // 16, carry=m, unroll=4)
            def _serve(k, mc):
                iv = idx_v[pl.ds(cb * OCHUNK + 16 * k, 16)]
                g = plsc.load_gather(trow_v, [iv])
                mc = jnp.maximum(mc, g * g)
                ob_v[buf, pl.ds(16 * k, 16)] = g * scale99
                return mc

            m = _serve
            out_cp[buf] = pltpu.async_copy(
                ob_v.at[buf], out_hbm.at[d, pl.ds(cb * OCHUNK, OCHUNK)],
                osem)

        if p + 1 < NPASS:
            for b in range(2):
                out_cp[b].wait()
                out_cp[b] = None
            stage = stage_row((p + 1) * NW + wid)

    for b in range(2):
        out_cp[b].wait()

    for c in range(8):
        ob_v[0, pl.ds(16 * c, 16)] = m if c == 0 else zero
    pltpu.sync_copy(ob_v.at[0, pl.ds(0, 128)], flags_hbm.at[wid])


def _fb_body(idx_hbm, table_hbm, out_hbm, idx_v, rows_v, sem):
    wid = lax.axis_index("s") * NC + lax.axis_index("c")
    bpw = B // NW
    pltpu.sync_copy(idx_hbm.at[wid], idx_v)
    copies = [
        pltpu.async_copy(
            table_hbm.at[idx_v.at[j]],
            rows_v.at[pl.ds(j * 128, 128)],
            sem,
        )
        for j in range(bpw // 128)
    ]
    for c in copies:
        c.wait()

    half = jnp.full((16,), -0.5, dtype=_F32)
    three_half = jnp.full((16,), 1.5, dtype=_F32)
    one = jnp.full((16,), 1.0, dtype=_F32)
    magic = jnp.full((16,), 0x5F3759DF, dtype=jnp.int32)
    scale99 = jnp.full((16,), 0.99, dtype=_F32)

    def row(i, carry):
        x0 = rows_v[i, pl.ds(0, 16)]
        x1 = rows_v[i, pl.ds(16, 16)]
        x2 = rows_v[i, pl.ds(32, 16)]
        x3 = rows_v[i, pl.ds(48, 16)]
        acc = x0 * x0 + x1 * x1 + x2 * x2 + x3 * x3
        nsq = jnp.maximum(jnp.full((16,), jnp.sum(acc), dtype=_F32), one)
        w = plsc.bitcast(magic - (plsc.bitcast(nsq, jnp.int32) >> 1), _F32)
        h = nsq * half
        w = w * (three_half + h * w * w)
        w = w * (three_half + h * w * w)
        w = w * scale99
        rows_v[i, pl.ds(0, 16)] = x0 * w
        rows_v[i, pl.ds(16, 16)] = x1 * w
        rows_v[i, pl.ds(32, 16)] = x2 * w
        rows_v[i, pl.ds(48, 16)] = x3 * w
        return carry

    lax.fori_loop(0, bpw, row, 0)
    pltpu.sync_copy(rows_v, out_hbm.at[pl.ds(wid * bpw, bpw)])


def _mesh():
    return plsc.VectorSubcoreMesh(core_axis_name="c", subcore_axis_name="s")


def _fallback(taxon_ids, table):
    idx = taxon_ids.astype(jnp.int32).reshape(NW, B // NW // 128, 128)
    k = pl.kernel(
        _fb_body,
        out_type=jax.ShapeDtypeStruct((B, D), _F32),
        mesh=_mesh(),
        compiler_params=pltpu.CompilerParams(
            needs_layout_passes=False, use_tc_tiling_on_sc=False),
        scratch_types=[
            pltpu.VMEM((B // NW // 128, 128), jnp.int32),
            pltpu.VMEM((B // NW, D), _F32),
            pltpu.SemaphoreType.DMA,
        ],
    )
    return k(idx, table)


def kernel(taxon_ids, table):
    table_t = table.T
    idx = taxon_ids.astype(jnp.int32)
    k = pl.kernel(
        _sc_body,
        out_type=(
            jax.ShapeDtypeStruct((D, B), _F32),
            jax.ShapeDtypeStruct((NW, 128), _F32),
        ),
        mesh=_mesh(),
        compiler_params=pltpu.CompilerParams(
            needs_layout_passes=False, use_tc_tiling_on_sc=True),
        scratch_types=[
            pltpu.VMEM((B,), jnp.int32),
            pltpu.VMEM((V,), _F32),
            pltpu.VMEM((2, OCHUNK), _F32),
            pltpu.SemaphoreType.DMA,
            pltpu.SemaphoreType.DMA,
            pltpu.SemaphoreType.DMA,
        ],
    )
    out_t, flags = k(table_t, idx)
    tripped = jnp.max(flags) * 64.0 > 1.0
    return lax.cond(tripped,
                    lambda: _fallback(taxon_ids, table),
                    lambda: out_t.T)
